# Initial kernel scaffold; baseline (speedup 1.0000x reference)
import jax
import jax.numpy as jnp
from jax.experimental import pallas as pl


def kernel(x, edge_index, W0, b0, W1, b1, W2, b2):
    raise NotImplementedError("write your pallas kernel here")


# speedup vs baseline: 10.8322x; 10.8322x over previous
import functools

import jax
import jax.numpy as jnp
from jax import lax
from jax.experimental import pallas as pl
from jax.experimental.pallas import tpu as pltpu
from jax.experimental.pallas import tpu_sc as plsc

_N = 10000
_E = 320000
_D = 128
_DLAST = 64
_NCLS = 40

_NC = 2
_NS = 16
_NT = _NC * _NS
_CH = 125
_NCHD = _E // _NT // _CH
_NCHP = _E // _NS // _CH
_RPT = 640
_NPAD = _NS * _RPT
_ZB = 128

_mesh = lambda: plsc.VectorSubcoreMesh(core_axis_name="c", subcore_axis_name="s")


def _make_deg_kernel():
  @functools.partial(
      pl.kernel,
      mesh=_mesh(),
      compiler_params=pltpu.CompilerParams(use_tc_tiling_on_sc=False),
      out_type=jax.ShapeDtypeStruct((_NC, _NPAD, 16), jnp.float32),
      scratch_types=[
          pltpu.VMEM((_NCHD, _CH), jnp.int32),
          pltpu.VMEM((_CH, 16), jnp.float32),
          pltpu.VMEM((_ZB, 16), jnp.float32),
          pltpu.VMEM_SHARED((_NPAD, 16), jnp.float32),
      ],
  )
  def deg_kernel(dst_hbm, out_hbm, dst_v, ones_v, zbuf_v, acc):
    c = lax.axis_index("c")
    s = lax.axis_index("s")
    wid = c * _NS + s
    pltpu.sync_copy(dst_hbm.at[wid], dst_v)

    one16 = jnp.ones((16,), jnp.float32)
    zero16 = jnp.zeros((16,), jnp.float32)

    def fill_ones(i, carry):
      ones_v[i, :] = one16
      return carry

    lax.fori_loop(0, _CH, fill_ones, 0)

    def fill_zeros(i, carry):
      zbuf_v[i, :] = zero16
      return carry

    lax.fori_loop(0, _ZB, fill_zeros, 0)

    for k in range(_RPT // _ZB):
      pltpu.sync_copy(zbuf_v, acc.at[pl.ds(s * _RPT + k * _ZB, _ZB)])
    plsc.subcore_barrier()

    def body(j, carry):
      pltpu.sync_copy(ones_v, acc.at[dst_v.at[j]], add=True)
      return carry

    lax.fori_loop(0, _NCHD, body, 0)
    plsc.subcore_barrier()
    pltpu.sync_copy(acc.at[pl.ds(s * _RPT, _RPT)],
                    out_hbm.at[c, pl.ds(s * _RPT, _RPT)])

  return deg_kernel


def _make_prop_kernel(dw):
  dwh = dw // 2

  @functools.partial(
      pl.kernel,
      mesh=_mesh(),
      compiler_params=pltpu.CompilerParams(use_tc_tiling_on_sc=False),
      out_type=jax.ShapeDtypeStruct((_NC, _NPAD, dwh), jnp.float32),
      scratch_types=[
          pltpu.VMEM((_NCHP, _CH), jnp.int32),
          pltpu.VMEM((_NCHP, _CH), jnp.int32),
          pltpu.VMEM((2, _CH, dwh), jnp.float32),
          pltpu.VMEM((_ZB, dwh), jnp.float32),
          pltpu.VMEM_SHARED((_NPAD, dwh), jnp.float32),
          pltpu.SemaphoreType.DMA,
          pltpu.SemaphoreType.DMA,
      ],
  )
  def prop_kernel(src_hbm, dst_hbm, u_hbm, out_hbm,
                  src_v, dst_v, rows_v, zbuf_v, acc, sem0, sem1):
    c = lax.axis_index("c")
    s = lax.axis_index("s")
    pltpu.sync_copy(src_hbm.at[c, s], src_v)
    pltpu.sync_copy(dst_hbm.at[s], dst_v)

    zero16 = jnp.zeros((16,), jnp.float32)

    def fill(i, carry):
      for k in range(dwh // 16):
        zbuf_v[i, pl.ds(k * 16, 16)] = zero16
      return carry

    lax.fori_loop(0, _ZB, fill, 0)

    for k in range(_RPT // _ZB):
      pltpu.sync_copy(zbuf_v, acc.at[pl.ds(s * _RPT + k * _ZB, _ZB)])
    plsc.subcore_barrier()

    pltpu.async_copy(u_hbm.at[src_v.at[0]], rows_v.at[0], sem0)

    def body(k, carry):
      j0 = 2 * k
      j1 = j0 + 1
      pltpu.async_copy(u_hbm.at[src_v.at[j1]], rows_v.at[1], sem1)
      pltpu.make_async_copy(u_hbm.at[src_v.at[j0]], rows_v.at[0], sem0).wait()
      pltpu.sync_copy(rows_v.at[0], acc.at[dst_v.at[j0]], add=True)

      @pl.when(k < _NCHP // 2 - 1)
      def _():
        pltpu.async_copy(u_hbm.at[src_v.at[j0 + 2]], rows_v.at[0], sem0)

      pltpu.make_async_copy(u_hbm.at[src_v.at[j1]], rows_v.at[1], sem1).wait()
      pltpu.sync_copy(rows_v.at[1], acc.at[dst_v.at[j1]], add=True)
      return carry

    lax.fori_loop(0, _NCHP // 2, body, 0)
    plsc.subcore_barrier()
    pltpu.sync_copy(acc.at[pl.ds(s * _RPT, _RPT)],
                    out_hbm.at[c, pl.ds(s * _RPT, _RPT)])

  return prop_kernel


_deg_call = _make_deg_kernel()
_prop128 = _make_prop_kernel(_D)
_prop64 = _make_prop_kernel(_DLAST)


_R = 2000
_GRID = _N // _R


def _tc1_body(parts_ref, x_ref, w_ref, u0_ref, norm_ref):
  deg = parts_ref[0, :, 0:1] + parts_ref[1, :, 0:1]
  norm = lax.rsqrt(jnp.maximum(deg, 1.0))
  t = jnp.dot(x_ref[...], w_ref[...],
              preferred_element_type=jnp.float32) * norm
  u0_ref[0] = t[:, : _D // 2]
  u0_ref[1] = t[:, _D // 2:]
  norm_ref[...] = norm


def _tc1(parts, x, w0):
  return pl.pallas_call(
      _tc1_body,
      grid=(_GRID,),
      in_specs=[
          pl.BlockSpec((_NC, _R, 16), lambda i: (0, i, 0)),
          pl.BlockSpec((_R, _D), lambda i: (i, 0)),
          pl.BlockSpec((_D, _D), lambda i: (0, 0)),
      ],
      out_specs=[
          pl.BlockSpec((_NC, _R, _D // 2), lambda i: (0, i, 0)),
          pl.BlockSpec((_R, 1), lambda i: (i, 0)),
      ],
      out_shape=[
          jax.ShapeDtypeStruct((_NC, _N, _D // 2), jnp.float32),
          jax.ShapeDtypeStruct((_N, 1), jnp.float32),
      ],
  )(parts, x, w0)


def _mid_body(agg_ref, norm_ref, b_ref, w_ref, out_ref):
  a = jnp.concatenate([agg_ref[0], agg_ref[1]], axis=1)
  n = norm_ref[...]
  h = jnp.maximum(a * n + b_ref[...], 0.0)
  t = jnp.dot(h, w_ref[...], preferred_element_type=jnp.float32) * n
  dwh = t.shape[1] // 2
  out_ref[0] = t[:, :dwh]
  out_ref[1] = t[:, dwh:]


def _tc_mid(agg, norm, b, w, dw_out):
  return pl.pallas_call(
      _mid_body,
      grid=(_GRID,),
      in_specs=[
          pl.BlockSpec((_NC, _R, _D // 2), lambda i: (0, i, 0)),
          pl.BlockSpec((_R, 1), lambda i: (i, 0)),
          pl.BlockSpec((1, _D), lambda i: (0, 0)),
          pl.BlockSpec((_D, dw_out), lambda i: (0, 0)),
      ],
      out_specs=pl.BlockSpec((_NC, _R, dw_out // 2), lambda i: (0, i, 0)),
      out_shape=jax.ShapeDtypeStruct((_NC, _N, dw_out // 2), jnp.float32),
  )(agg, norm, b, w)


def _fin_body(agg_ref, norm_ref, b_ref, out_ref):
  a = jnp.concatenate([agg_ref[0], agg_ref[1]], axis=1)
  out_ref[...] = a * norm_ref[...] + b_ref[...]


def _tc_fin(agg, norm, b):
  return pl.pallas_call(
      _fin_body,
      grid=(_GRID,),
      in_specs=[
          pl.BlockSpec((_NC, _R, _DLAST // 2), lambda i: (0, i, 0)),
          pl.BlockSpec((_R, 1), lambda i: (i, 0)),
          pl.BlockSpec((1, _DLAST), lambda i: (0, 0)),
      ],
      out_specs=pl.BlockSpec((_R, _DLAST), lambda i: (i, 0)),
      out_shape=jax.ShapeDtypeStruct((_N, _DLAST), jnp.float32),
  )(agg, norm, b)


def kernel(x, edge_index, W0, b0, W1, b1, W2, b2):
  src = edge_index[0]
  dst = edge_index[1]

  dst32 = dst.reshape(_NT, _NCHD, _CH)
  src16 = src.reshape(_NS, _NCHP, _CH)
  src2 = jnp.stack([src16, src16 + _N])
  dst16 = dst.reshape(_NS, _NCHP, _CH)

  w2p = jnp.zeros((_D, _DLAST), jnp.float32).at[:, :_NCLS].set(W2)
  b2p = jnp.zeros((_DLAST,), jnp.float32).at[:_NCLS].set(b2)

  deg_parts = _deg_call(dst32)
  u0, norm = _tc1(deg_parts, x, W0)
  agg0 = _prop128(src2, dst16, u0.reshape(_NC * _N, _D // 2))
  u1 = _tc_mid(agg0, norm, b0.reshape(1, _D), W1, _D)
  agg1 = _prop128(src2, dst16, u1.reshape(_NC * _N, _D // 2))
  u2 = _tc_mid(agg1, norm, b1.reshape(1, _D), w2p, _DLAST)
  agg2 = _prop64(src2, dst16, u2.reshape(_NC * _N, _DLAST // 2))
  out = _tc_fin(agg2, norm, b2p.reshape(1, _DLAST))
  return out[:, :_NCLS]


# speedup vs baseline: 12.6118x; 1.1643x over previous
import functools

import jax
import jax.numpy as jnp
from jax import lax
from jax.experimental import pallas as pl
from jax.experimental.pallas import tpu as pltpu
from jax.experimental.pallas import tpu_sc as plsc

_N = 10000
_E = 320000
_D = 128
_DLAST = 64
_NCLS = 40

_NC = 2
_NS = 16
_NT = _NC * _NS
_CH = 125
_CHP = 250
_NCHD = _E // _NT // _CH
_NCHP = _E // _NS // _CHP
_RPT = 640
_NPAD = _NS * _RPT
_ZB = 128

_mesh = lambda: plsc.VectorSubcoreMesh(core_axis_name="c", subcore_axis_name="s")


def _make_deg_kernel():
  @functools.partial(
      pl.kernel,
      mesh=_mesh(),
      compiler_params=pltpu.CompilerParams(use_tc_tiling_on_sc=False),
      out_type=jax.ShapeDtypeStruct((_NC, _NPAD, 16), jnp.float32),
      scratch_types=[
          pltpu.VMEM((_NCHD, _CH), jnp.int32),
          pltpu.VMEM((_CH, 16), jnp.float32),
          pltpu.VMEM((_ZB, 16), jnp.float32),
          pltpu.VMEM_SHARED((_NPAD, 16), jnp.float32),
      ],
  )
  def deg_kernel(dst_hbm, out_hbm, dst_v, ones_v, zbuf_v, acc):
    c = lax.axis_index("c")
    s = lax.axis_index("s")
    wid = c * _NS + s
    pltpu.sync_copy(dst_hbm.at[wid], dst_v)

    one16 = jnp.ones((16,), jnp.float32)
    zero16 = jnp.zeros((16,), jnp.float32)

    def fill_ones(i, carry):
      ones_v[i, :] = one16
      return carry

    lax.fori_loop(0, _CH, fill_ones, 0)

    def fill_zeros(i, carry):
      zbuf_v[i, :] = zero16
      return carry

    lax.fori_loop(0, _ZB, fill_zeros, 0)

    for k in range(_RPT // _ZB):
      pltpu.sync_copy(zbuf_v, acc.at[pl.ds(s * _RPT + k * _ZB, _ZB)])
    plsc.subcore_barrier()

    def body(j, carry):
      pltpu.sync_copy(ones_v, acc.at[dst_v.at[j]], add=True)
      return carry

    lax.fori_loop(0, _NCHD, body, 0)
    plsc.subcore_barrier()
    pltpu.sync_copy(acc.at[pl.ds(s * _RPT, _RPT)],
                    out_hbm.at[c, pl.ds(s * _RPT, _RPT)])

  return deg_kernel


def _make_prop_kernel(dw):
  dwh = dw // 2

  @functools.partial(
      pl.kernel,
      mesh=_mesh(),
      compiler_params=pltpu.CompilerParams(use_tc_tiling_on_sc=False),
      out_type=jax.ShapeDtypeStruct((_NC, _NPAD, dwh), jnp.float32),
      scratch_types=[
          pltpu.VMEM((_NCHP, _CHP), jnp.int32),
          pltpu.VMEM((_NCHP, _CHP), jnp.int32),
          pltpu.VMEM((2, _CHP, dwh), jnp.float32),
          pltpu.VMEM((_ZB, dwh), jnp.float32),
          pltpu.VMEM_SHARED((_NPAD, dwh), jnp.float32),
          pltpu.SemaphoreType.DMA,
          pltpu.SemaphoreType.DMA,
      ],
  )
  def prop_kernel(src_hbm, dst_hbm, u_hbm, out_hbm,
                  src_v, dst_v, rows_v, zbuf_v, acc, sem0, sem1):
    c = lax.axis_index("c")
    s = lax.axis_index("s")
    pltpu.sync_copy(src_hbm.at[c, s], src_v)
    pltpu.sync_copy(dst_hbm.at[s], dst_v)

    zero16 = jnp.zeros((16,), jnp.float32)

    def fill(i, carry):
      for k in range(dwh // 16):
        zbuf_v[i, pl.ds(k * 16, 16)] = zero16
      return carry

    lax.fori_loop(0, _ZB, fill, 0)

    for k in range(_RPT // _ZB):
      pltpu.sync_copy(zbuf_v, acc.at[pl.ds(s * _RPT + k * _ZB, _ZB)])
    plsc.subcore_barrier()

    pltpu.async_copy(u_hbm.at[src_v.at[0]], rows_v.at[0], sem0)

    def body(k, carry):
      j0 = 2 * k
      j1 = j0 + 1
      pltpu.async_copy(u_hbm.at[src_v.at[j1]], rows_v.at[1], sem1)
      pltpu.make_async_copy(u_hbm.at[src_v.at[j0]], rows_v.at[0], sem0).wait()
      pltpu.sync_copy(rows_v.at[0], acc.at[dst_v.at[j0]], add=True)

      @pl.when(k < _NCHP // 2 - 1)
      def _():
        pltpu.async_copy(u_hbm.at[src_v.at[j0 + 2]], rows_v.at[0], sem0)

      pltpu.make_async_copy(u_hbm.at[src_v.at[j1]], rows_v.at[1], sem1).wait()
      pltpu.sync_copy(rows_v.at[1], acc.at[dst_v.at[j1]], add=True)
      return carry

    lax.fori_loop(0, _NCHP // 2, body, 0)
    plsc.subcore_barrier()
    pltpu.sync_copy(acc.at[pl.ds(s * _RPT, _RPT)],
                    out_hbm.at[c, pl.ds(s * _RPT, _RPT)])

  return prop_kernel


_deg_call = _make_deg_kernel()
_prop128 = _make_prop_kernel(_D)
_prop64 = _make_prop_kernel(_DLAST)


_R = 2000
_GRID = _N // _R


def _tc1_body(parts_ref, x_ref, w_ref, u0_ref, norm_ref):
  deg = parts_ref[0, :, 0:1] + parts_ref[1, :, 0:1]
  norm = lax.rsqrt(jnp.maximum(deg, 1.0))
  t = jnp.dot(x_ref[...], w_ref[...],
              preferred_element_type=jnp.float32) * norm
  u0_ref[0] = t[:, : _D // 2]
  u0_ref[1] = t[:, _D // 2:]
  norm_ref[...] = norm


def _tc1(parts, x, w0):
  return pl.pallas_call(
      _tc1_body,
      grid=(_GRID,),
      in_specs=[
          pl.BlockSpec((_NC, _R, 16), lambda i: (0, i, 0)),
          pl.BlockSpec((_R, _D), lambda i: (i, 0)),
          pl.BlockSpec((_D, _D), lambda i: (0, 0)),
      ],
      out_specs=[
          pl.BlockSpec((_NC, _R, _D // 2), lambda i: (0, i, 0)),
          pl.BlockSpec((_R, 1), lambda i: (i, 0)),
      ],
      out_shape=[
          jax.ShapeDtypeStruct((_NC, _N, _D // 2), jnp.float32),
          jax.ShapeDtypeStruct((_N, 1), jnp.float32),
      ],
  )(parts, x, w0)


def _mid_body(agg_ref, norm_ref, b_ref, w_ref, out_ref):
  a = jnp.concatenate([agg_ref[0], agg_ref[1]], axis=1)
  n = norm_ref[...]
  h = jnp.maximum(a * n + b_ref[...], 0.0)
  t = jnp.dot(h, w_ref[...], preferred_element_type=jnp.float32) * n
  dwh = t.shape[1] // 2
  out_ref[0] = t[:, :dwh]
  out_ref[1] = t[:, dwh:]


def _tc_mid(agg, norm, b, w, dw_out):
  return pl.pallas_call(
      _mid_body,
      grid=(_GRID,),
      in_specs=[
          pl.BlockSpec((_NC, _R, _D // 2), lambda i: (0, i, 0)),
          pl.BlockSpec((_R, 1), lambda i: (i, 0)),
          pl.BlockSpec((1, _D), lambda i: (0, 0)),
          pl.BlockSpec((_D, dw_out), lambda i: (0, 0)),
      ],
      out_specs=pl.BlockSpec((_NC, _R, dw_out // 2), lambda i: (0, i, 0)),
      out_shape=jax.ShapeDtypeStruct((_NC, _N, dw_out // 2), jnp.float32),
  )(agg, norm, b, w)


def _fin_body(agg_ref, norm_ref, b_ref, out_ref):
  a = jnp.concatenate([agg_ref[0], agg_ref[1]], axis=1)
  out_ref[...] = a * norm_ref[...] + b_ref[...]


def _tc_fin(agg, norm, b):
  return pl.pallas_call(
      _fin_body,
      grid=(_GRID,),
      in_specs=[
          pl.BlockSpec((_NC, _R, _DLAST // 2), lambda i: (0, i, 0)),
          pl.BlockSpec((_R, 1), lambda i: (i, 0)),
          pl.BlockSpec((1, _DLAST), lambda i: (0, 0)),
      ],
      out_specs=pl.BlockSpec((_R, _DLAST), lambda i: (i, 0)),
      out_shape=jax.ShapeDtypeStruct((_N, _DLAST), jnp.float32),
  )(agg, norm, b)


def kernel(x, edge_index, W0, b0, W1, b1, W2, b2):
  src = edge_index[0]
  dst = edge_index[1]

  dst32 = dst.reshape(_NT, _NCHD, _CH)
  src16 = src.reshape(_NS, _NCHP, _CHP)
  src2 = jnp.stack([src16, src16 + _N])
  dst16 = dst.reshape(_NS, _NCHP, _CHP)

  w2p = jnp.zeros((_D, _DLAST), jnp.float32).at[:, :_NCLS].set(W2)
  b2p = jnp.zeros((_DLAST,), jnp.float32).at[:_NCLS].set(b2)

  deg_parts = _deg_call(dst32)
  u0, norm = _tc1(deg_parts, x, W0)
  agg0 = _prop128(src2, dst16, u0.reshape(_NC * _N, _D // 2))
  u1 = _tc_mid(agg0, norm, b0.reshape(1, _D), W1, _D)
  agg1 = _prop128(src2, dst16, u1.reshape(_NC * _N, _D // 2))
  u2 = _tc_mid(agg1, norm, b1.reshape(1, _D), w2p, _DLAST)
  agg2 = _prop64(src2, dst16, u2.reshape(_NC * _N, _DLAST // 2))
  out = _tc_fin(agg2, norm, b2p.reshape(1, _DLAST))
  return out[:, :_NCLS]


# speedup vs baseline: 12.6771x; 1.0052x over previous
import functools

import jax
import jax.numpy as jnp
from jax import lax
from jax.experimental import pallas as pl
from jax.experimental.pallas import tpu as pltpu
from jax.experimental.pallas import tpu_sc as plsc

_N = 10000
_E = 320000
_D = 128
_DLAST = 64
_NCLS = 40

_NC = 2
_NS = 16
_NT = _NC * _NS
_CH = 250
_CHP = 250
_NCHD = _E // _NT // _CH
_NCHP = _E // _NS // _CHP
_RPT = 640
_NPAD = _NS * _RPT
_ZB = 128

_mesh = lambda: plsc.VectorSubcoreMesh(core_axis_name="c", subcore_axis_name="s")


def _make_deg_kernel():
  @functools.partial(
      pl.kernel,
      mesh=_mesh(),
      compiler_params=pltpu.CompilerParams(use_tc_tiling_on_sc=False),
      out_type=jax.ShapeDtypeStruct((_NC, _NPAD, 16), jnp.float32),
      scratch_types=[
          pltpu.VMEM((_NCHD, _CH), jnp.int32),
          pltpu.VMEM((_CH, 16), jnp.float32),
          pltpu.VMEM((_ZB, 16), jnp.float32),
          pltpu.VMEM_SHARED((_NPAD, 16), jnp.float32),
      ],
  )
  def deg_kernel(dst_hbm, out_hbm, dst_v, ones_v, zbuf_v, acc):
    c = lax.axis_index("c")
    s = lax.axis_index("s")
    wid = c * _NS + s
    pltpu.sync_copy(dst_hbm.at[wid], dst_v)

    one16 = jnp.ones((16,), jnp.float32)
    zero16 = jnp.zeros((16,), jnp.float32)

    def fill_ones(i, carry):
      ones_v[i, :] = one16
      return carry

    lax.fori_loop(0, _CH, fill_ones, 0)

    def fill_zeros(i, carry):
      zbuf_v[i, :] = zero16
      return carry

    lax.fori_loop(0, _ZB, fill_zeros, 0)

    for k in range(_RPT // _ZB):
      pltpu.sync_copy(zbuf_v, acc.at[pl.ds(s * _RPT + k * _ZB, _ZB)])
    plsc.subcore_barrier()

    def body(j, carry):
      pltpu.sync_copy(ones_v, acc.at[dst_v.at[j]], add=True)
      return carry

    lax.fori_loop(0, _NCHD, body, 0)
    plsc.subcore_barrier()
    pltpu.sync_copy(acc.at[pl.ds(s * _RPT, _RPT)],
                    out_hbm.at[c, pl.ds(s * _RPT, _RPT)])

  return deg_kernel


def _make_prop_kernel(dw):
  dwh = dw // 2

  @functools.partial(
      pl.kernel,
      mesh=_mesh(),
      compiler_params=pltpu.CompilerParams(use_tc_tiling_on_sc=False),
      out_type=jax.ShapeDtypeStruct((_NC, _NPAD, dwh), jnp.float32),
      scratch_types=[
          pltpu.VMEM((_NCHP, _CHP), jnp.int32),
          pltpu.VMEM((_NCHP, _CHP), jnp.int32),
          pltpu.VMEM((2, _CHP, dwh), jnp.float32),
          pltpu.VMEM((_ZB, dwh), jnp.float32),
          pltpu.VMEM_SHARED((_NPAD, dwh), jnp.float32),
          pltpu.SemaphoreType.DMA,
          pltpu.SemaphoreType.DMA,
      ],
  )
  def prop_kernel(src_hbm, dst_hbm, u_hbm, out_hbm,
                  src_v, dst_v, rows_v, zbuf_v, acc, sem0, sem1):
    c = lax.axis_index("c")
    s = lax.axis_index("s")
    pltpu.sync_copy(src_hbm.at[c, s], src_v)
    pltpu.sync_copy(dst_hbm.at[s], dst_v)

    zero16 = jnp.zeros((16,), jnp.float32)

    def fill(i, carry):
      for k in range(dwh // 16):
        zbuf_v[i, pl.ds(k * 16, 16)] = zero16
      return carry

    lax.fori_loop(0, _ZB, fill, 0)

    for k in range(_RPT // _ZB):
      pltpu.sync_copy(zbuf_v, acc.at[pl.ds(s * _RPT + k * _ZB, _ZB)])
    plsc.subcore_barrier()

    pltpu.async_copy(u_hbm.at[src_v.at[0]], rows_v.at[0], sem0)

    def body(k, carry):
      j0 = 2 * k
      j1 = j0 + 1
      pltpu.async_copy(u_hbm.at[src_v.at[j1]], rows_v.at[1], sem1)
      pltpu.make_async_copy(u_hbm.at[src_v.at[j0]], rows_v.at[0], sem0).wait()
      pltpu.sync_copy(rows_v.at[0], acc.at[dst_v.at[j0]], add=True)

      @pl.when(k < _NCHP // 2 - 1)
      def _():
        pltpu.async_copy(u_hbm.at[src_v.at[j0 + 2]], rows_v.at[0], sem0)

      pltpu.make_async_copy(u_hbm.at[src_v.at[j1]], rows_v.at[1], sem1).wait()
      pltpu.sync_copy(rows_v.at[1], acc.at[dst_v.at[j1]], add=True)
      return carry

    lax.fori_loop(0, _NCHP // 2, body, 0)
    plsc.subcore_barrier()
    pltpu.sync_copy(acc.at[pl.ds(s * _RPT, _RPT)],
                    out_hbm.at[c, pl.ds(s * _RPT, _RPT)])

  return prop_kernel


_deg_call = _make_deg_kernel()
_prop128 = _make_prop_kernel(_D)
_prop64 = _make_prop_kernel(_DLAST)


_R = 2000
_GRID = _N // _R


def _tc1_body(parts_ref, x_ref, w_ref, u0_ref, norm_ref):
  deg = parts_ref[0, :, 0:1] + parts_ref[1, :, 0:1]
  norm = lax.rsqrt(jnp.maximum(deg, 1.0))
  t = jnp.dot(x_ref[...], w_ref[...],
              preferred_element_type=jnp.float32) * norm
  u0_ref[0] = t[:, : _D // 2]
  u0_ref[1] = t[:, _D // 2:]
  norm_ref[...] = norm


def _tc1(parts, x, w0):
  return pl.pallas_call(
      _tc1_body,
      grid=(_GRID,),
      in_specs=[
          pl.BlockSpec((_NC, _R, 16), lambda i: (0, i, 0)),
          pl.BlockSpec((_R, _D), lambda i: (i, 0)),
          pl.BlockSpec((_D, _D), lambda i: (0, 0)),
      ],
      out_specs=[
          pl.BlockSpec((_NC, _R, _D // 2), lambda i: (0, i, 0)),
          pl.BlockSpec((_R, 1), lambda i: (i, 0)),
      ],
      out_shape=[
          jax.ShapeDtypeStruct((_NC, _N, _D // 2), jnp.float32),
          jax.ShapeDtypeStruct((_N, 1), jnp.float32),
      ],
  )(parts, x, w0)


def _mid_body(agg_ref, norm_ref, b_ref, w_ref, out_ref):
  a = jnp.concatenate([agg_ref[0], agg_ref[1]], axis=1)
  n = norm_ref[...]
  h = jnp.maximum(a * n + b_ref[...], 0.0)
  t = jnp.dot(h, w_ref[...], preferred_element_type=jnp.float32) * n
  dwh = t.shape[1] // 2
  out_ref[0] = t[:, :dwh]
  out_ref[1] = t[:, dwh:]


def _tc_mid(agg, norm, b, w, dw_out):
  return pl.pallas_call(
      _mid_body,
      grid=(_GRID,),
      in_specs=[
          pl.BlockSpec((_NC, _R, _D // 2), lambda i: (0, i, 0)),
          pl.BlockSpec((_R, 1), lambda i: (i, 0)),
          pl.BlockSpec((1, _D), lambda i: (0, 0)),
          pl.BlockSpec((_D, dw_out), lambda i: (0, 0)),
      ],
      out_specs=pl.BlockSpec((_NC, _R, dw_out // 2), lambda i: (0, i, 0)),
      out_shape=jax.ShapeDtypeStruct((_NC, _N, dw_out // 2), jnp.float32),
  )(agg, norm, b, w)


def _fin_body(agg_ref, norm_ref, b_ref, out_ref):
  a = jnp.concatenate([agg_ref[0], agg_ref[1]], axis=1)
  out_ref[...] = a * norm_ref[...] + b_ref[...]


def _tc_fin(agg, norm, b):
  return pl.pallas_call(
      _fin_body,
      grid=(_GRID,),
      in_specs=[
          pl.BlockSpec((_NC, _R, _DLAST // 2), lambda i: (0, i, 0)),
          pl.BlockSpec((_R, 1), lambda i: (i, 0)),
          pl.BlockSpec((1, _DLAST), lambda i: (0, 0)),
      ],
      out_specs=pl.BlockSpec((_R, _DLAST), lambda i: (i, 0)),
      out_shape=jax.ShapeDtypeStruct((_N, _DLAST), jnp.float32),
  )(agg, norm, b)


def kernel(x, edge_index, W0, b0, W1, b1, W2, b2):
  src = edge_index[0]
  dst = edge_index[1]

  dst32 = dst.reshape(_NT, _NCHD, _CH)
  src16 = src.reshape(_NS, _NCHP, _CHP)
  src2 = jnp.stack([src16, src16 + _N])
  dst16 = dst.reshape(_NS, _NCHP, _CHP)

  w2p = jnp.zeros((_D, _DLAST), jnp.float32).at[:, :_NCLS].set(W2)
  b2p = jnp.zeros((_DLAST,), jnp.float32).at[:_NCLS].set(b2)

  deg_parts = _deg_call(dst32)
  u0, norm = _tc1(deg_parts, x, W0)
  agg0 = _prop128(src2, dst16, u0.reshape(_NC * _N, _D // 2))
  u1 = _tc_mid(agg0, norm, b0.reshape(1, _D), W1, _D)
  agg1 = _prop128(src2, dst16, u1.reshape(_NC * _N, _D // 2))
  u2 = _tc_mid(agg1, norm, b1.reshape(1, _D), w2p, _DLAST)
  agg2 = _prop64(src2, dst16, u2.reshape(_NC * _N, _DLAST // 2))
  out = _tc_fin(agg2, norm, b2p.reshape(1, _DLAST))
  return out[:, :_NCLS]


# speedup vs baseline: 13.8542x; 1.0929x over previous
import functools

import jax
import jax.numpy as jnp
from jax import lax
from jax.experimental import pallas as pl
from jax.experimental.pallas import tpu as pltpu
from jax.experimental.pallas import tpu_sc as plsc

_N = 10000
_E = 320000
_D = 128
_DLAST = 64
_NCLS = 40

_NC = 2
_NS = 16
_NT = _NC * _NS
_CHP = 200
_EPD = _E // _NT
_EPP = _E // _NS
_NCHD = _EPD // _CHP
_NCHP = _EPP // _CHP
_RPT = 640
_NPAD = _NS * _RPT
_ZB = 128

_mesh = lambda: plsc.VectorSubcoreMesh(core_axis_name="c", subcore_axis_name="s")


def _make_deg_kernel():
  @functools.partial(
      pl.kernel,
      mesh=_mesh(),
      compiler_params=pltpu.CompilerParams(use_tc_tiling_on_sc=False),
      out_type=jax.ShapeDtypeStruct((_NC, _NPAD, 16), jnp.float32),
      scratch_types=[
          pltpu.VMEM((_EPD,), jnp.int32),
          pltpu.VMEM((_CHP, 16), jnp.float32),
          pltpu.VMEM((_ZB, 16), jnp.float32),
          pltpu.VMEM_SHARED((_NPAD, 16), jnp.float32),
      ],
  )
  def deg_kernel(dst_hbm, out_hbm, dst_v, ones_v, zbuf_v, acc):
    c = lax.axis_index("c")
    s = lax.axis_index("s")
    wid = c * _NS + s
    pltpu.sync_copy(dst_hbm.at[pl.ds(wid * _EPD, _EPD)], dst_v)

    one16 = jnp.ones((16,), jnp.float32)
    zero16 = jnp.zeros((16,), jnp.float32)

    def fill_ones(i, carry):
      ones_v[i, :] = one16
      return carry

    lax.fori_loop(0, _CHP, fill_ones, 0)

    def fill_zeros(i, carry):
      zbuf_v[i, :] = zero16
      return carry

    lax.fori_loop(0, _ZB, fill_zeros, 0)

    for k in range(_RPT // _ZB):
      pltpu.sync_copy(zbuf_v, acc.at[pl.ds(s * _RPT + k * _ZB, _ZB)])
    plsc.subcore_barrier()

    def body(j, carry):
      pltpu.sync_copy(ones_v, acc.at[dst_v.at[pl.ds(j * _CHP, _CHP)]], add=True)
      return carry

    lax.fori_loop(0, _NCHD, body, 0)
    plsc.subcore_barrier()
    pltpu.sync_copy(acc.at[pl.ds(s * _RPT, _RPT)],
                    out_hbm.at[c, pl.ds(s * _RPT, _RPT)])

  return deg_kernel


def _make_prop_kernel(dw):
  dwh = dw // 2

  @functools.partial(
      pl.kernel,
      mesh=_mesh(),
      compiler_params=pltpu.CompilerParams(use_tc_tiling_on_sc=False),
      out_type=jax.ShapeDtypeStruct((_NPAD, dw), jnp.float32),
      scratch_types=[
          pltpu.VMEM((_EPP,), jnp.int32),
          pltpu.VMEM((_EPP,), jnp.int32),
          pltpu.VMEM((2, _CHP, dwh), jnp.float32),
          pltpu.VMEM((_ZB, dwh), jnp.float32),
          pltpu.VMEM_SHARED((_NPAD, dwh), jnp.float32),
          pltpu.SemaphoreType.DMA,
          pltpu.SemaphoreType.DMA,
      ],
  )
  def prop_kernel(src_hbm, dst_hbm, u_hbm, out_hbm,
                  src_v, dst_v, rows_v, zbuf_v, acc, sem0, sem1):
    c = lax.axis_index("c")
    s = lax.axis_index("s")
    pltpu.sync_copy(src_hbm.at[pl.ds(c * _E + s * _EPP, _EPP)], src_v)
    pltpu.sync_copy(dst_hbm.at[pl.ds(s * _EPP, _EPP)], dst_v)

    zero16 = jnp.zeros((16,), jnp.float32)

    def fill(i, carry):
      for k in range(dwh // 16):
        zbuf_v[i, pl.ds(k * 16, 16)] = zero16
      return carry

    lax.fori_loop(0, _ZB, fill, 0)

    for k in range(_RPT // _ZB):
      pltpu.sync_copy(zbuf_v, acc.at[pl.ds(s * _RPT + k * _ZB, _ZB)])
    plsc.subcore_barrier()

    def _src(j):
      return src_v.at[pl.ds(j * _CHP, _CHP)]

    def _dst(j):
      return dst_v.at[pl.ds(j * _CHP, _CHP)]

    pltpu.async_copy(u_hbm.at[_src(0)], rows_v.at[0], sem0)

    def body(k, carry):
      j0 = 2 * k
      j1 = j0 + 1
      pltpu.async_copy(u_hbm.at[_src(j1)], rows_v.at[1], sem1)
      pltpu.make_async_copy(u_hbm.at[_src(j0)], rows_v.at[0], sem0).wait()
      pltpu.sync_copy(rows_v.at[0], acc.at[_dst(j0)], add=True)

      @pl.when(k < _NCHP // 2 - 1)
      def _():
        pltpu.async_copy(u_hbm.at[_src(j0 + 2)], rows_v.at[0], sem0)

      pltpu.make_async_copy(u_hbm.at[_src(j1)], rows_v.at[1], sem1).wait()
      pltpu.sync_copy(rows_v.at[1], acc.at[_dst(j1)], add=True)
      return carry

    lax.fori_loop(0, _NCHP // 2, body, 0)
    plsc.subcore_barrier()
    pltpu.sync_copy(acc.at[pl.ds(s * _RPT, _RPT)],
                    out_hbm.at[pl.ds(s * _RPT, _RPT), pl.ds(c * dwh, dwh)])

  return prop_kernel


_deg_call = _make_deg_kernel()
_prop128 = _make_prop_kernel(_D)
_prop64 = _make_prop_kernel(_DLAST)


_R = 2000
_GRID = _N // _R


def _tc1_body(parts_ref, x_ref, w_ref, u0_ref, norm_ref):
  deg = parts_ref[0, :, 0:1] + parts_ref[1, :, 0:1]
  norm = lax.rsqrt(jnp.maximum(deg, 1.0))
  t = jnp.dot(x_ref[...], w_ref[...],
              preferred_element_type=jnp.float32) * norm
  u0_ref[...] = t
  norm_ref[...] = norm


def _tc1(parts, x, w0):
  return pl.pallas_call(
      _tc1_body,
      grid=(_GRID,),
      in_specs=[
          pl.BlockSpec((_NC, _R, 16), lambda i: (0, i, 0)),
          pl.BlockSpec((_R, _D), lambda i: (i, 0)),
          pl.BlockSpec((_D, _D), lambda i: (0, 0)),
      ],
      out_specs=[
          pl.BlockSpec((_R, _D), lambda i: (i, 0)),
          pl.BlockSpec((_R, 1), lambda i: (i, 0)),
      ],
      out_shape=[
          jax.ShapeDtypeStruct((_N, _D), jnp.float32),
          jax.ShapeDtypeStruct((_N, 1), jnp.float32),
      ],
  )(parts, x, w0)


def _mid_body(agg_ref, norm_ref, b_ref, w_ref, out_ref):
  n = norm_ref[...]
  h = jnp.maximum(agg_ref[...] * n + b_ref[...], 0.0)
  out_ref[...] = jnp.dot(h, w_ref[...], preferred_element_type=jnp.float32) * n


def _tc_mid(agg, norm, b, w, dw_out):
  return pl.pallas_call(
      _mid_body,
      grid=(_GRID,),
      in_specs=[
          pl.BlockSpec((_R, _D), lambda i: (i, 0)),
          pl.BlockSpec((_R, 1), lambda i: (i, 0)),
          pl.BlockSpec((1, _D), lambda i: (0, 0)),
          pl.BlockSpec((_D, dw_out), lambda i: (0, 0)),
      ],
      out_specs=pl.BlockSpec((_R, dw_out), lambda i: (i, 0)),
      out_shape=jax.ShapeDtypeStruct((_N, dw_out), jnp.float32),
  )(agg, norm, b, w)


def _fin_body(agg_ref, norm_ref, b_ref, out_ref):
  out_ref[...] = agg_ref[...] * norm_ref[...] + b_ref[...]


def _tc_fin(agg, norm, b):
  return pl.pallas_call(
      _fin_body,
      grid=(_GRID,),
      in_specs=[
          pl.BlockSpec((_R, _DLAST), lambda i: (i, 0)),
          pl.BlockSpec((_R, 1), lambda i: (i, 0)),
          pl.BlockSpec((1, _DLAST), lambda i: (0, 0)),
      ],
      out_specs=pl.BlockSpec((_R, _DLAST), lambda i: (i, 0)),
      out_shape=jax.ShapeDtypeStruct((_N, _DLAST), jnp.float32),
  )(agg, norm, b)


def kernel(x, edge_index, W0, b0, W1, b1, W2, b2):
  src = edge_index[0]
  dst = edge_index[1]

  src2x = jnp.concatenate([src * 2, src * 2 + 1])
  dstf = dst.reshape(_E)

  w2p = jnp.zeros((_D, _DLAST), jnp.float32).at[:, :_NCLS].set(W2)
  b2p = jnp.zeros((_DLAST,), jnp.float32).at[:_NCLS].set(b2)

  deg_parts = _deg_call(dstf)
  u0, norm = _tc1(deg_parts, x, W0)
  agg0 = _prop128(src2x, dstf, u0.reshape(2 * _N, _D // 2))
  u1 = _tc_mid(agg0, norm,
               b0.reshape(1, _D), W1, _D)
  agg1 = _prop128(src2x, dstf, u1.reshape(2 * _N, _D // 2))
  u2 = _tc_mid(agg1, norm,
               b1.reshape(1, _D), w2p, _DLAST)
  agg2 = _prop64(src2x, dstf, u2.reshape(2 * _N, _DLAST // 2))
  out = _tc_fin(agg2, norm,
                b2p.reshape(1, _DLAST))
  return out[:, :_NCLS]


# speedup vs baseline: 15.7526x; 1.1370x over previous
import functools

import jax
import jax.numpy as jnp
from jax import lax
from jax.experimental import pallas as pl
from jax.experimental.pallas import tpu as pltpu
from jax.experimental.pallas import tpu_sc as plsc

_N = 10000
_E = 320000
_D = 128
_DLAST = 64
_NCLS = 40

_NC = 2
_NS = 16
_NT = _NC * _NS
_CHP = 200
_EPD = _E // _NT
_EPP = _E // _NS
_NCHD = _EPD // _CHP
_NCHP = _EPP // _CHP
_RPT = 640
_NPAD = _NS * _RPT
_ZB = 128

_mesh = lambda: plsc.VectorSubcoreMesh(core_axis_name="c", subcore_axis_name="s")


def _make_deg_kernel():
  @functools.partial(
      pl.kernel,
      mesh=_mesh(),
      compiler_params=pltpu.CompilerParams(use_tc_tiling_on_sc=False),
      out_type=jax.ShapeDtypeStruct((_NC, _NPAD, 16), jnp.float32),
      scratch_types=[
          pltpu.VMEM((_EPD,), jnp.int32),
          pltpu.VMEM((_CHP, 16), jnp.float32),
          pltpu.VMEM((_ZB, 16), jnp.float32),
          pltpu.VMEM_SHARED((_NPAD, 16), jnp.float32),
      ],
  )
  def deg_kernel(dst_hbm, out_hbm, dst_v, ones_v, zbuf_v, acc):
    c = lax.axis_index("c")
    s = lax.axis_index("s")
    wid = c * _NS + s
    pltpu.sync_copy(dst_hbm.at[pl.ds(wid * _EPD, _EPD)], dst_v)

    one16 = jnp.ones((16,), jnp.float32)
    zero16 = jnp.zeros((16,), jnp.float32)

    def fill_ones(i, carry):
      ones_v[i, :] = one16
      return carry

    lax.fori_loop(0, _CHP, fill_ones, 0)

    def fill_zeros(i, carry):
      zbuf_v[i, :] = zero16
      return carry

    lax.fori_loop(0, _ZB, fill_zeros, 0)

    for k in range(_RPT // _ZB):
      pltpu.sync_copy(zbuf_v, acc.at[pl.ds(s * _RPT + k * _ZB, _ZB)])
    plsc.subcore_barrier()

    def body(j, carry):
      pltpu.sync_copy(ones_v, acc.at[dst_v.at[pl.ds(j * _CHP, _CHP)]], add=True)
      return carry

    lax.fori_loop(0, _NCHD, body, 0)
    plsc.subcore_barrier()
    pltpu.sync_copy(acc.at[pl.ds(s * _RPT, _RPT)],
                    out_hbm.at[c, pl.ds(s * _RPT, _RPT)])

  return deg_kernel


def _make_prop_kernel(dw):
  dwh = dw // 2

  @functools.partial(
      pl.kernel,
      mesh=_mesh(),
      compiler_params=pltpu.CompilerParams(use_tc_tiling_on_sc=False),
      out_type=jax.ShapeDtypeStruct((_NPAD, dw), jnp.float32),
      scratch_types=[
          pltpu.VMEM((_EPP,), jnp.int32),
          pltpu.VMEM((_EPP,), jnp.int32),
          pltpu.VMEM((3, _CHP, dwh), jnp.float32),
          pltpu.VMEM_SHARED((_NPAD, dwh), jnp.float32),
          pltpu.SemaphoreType.DMA,
          pltpu.SemaphoreType.DMA,
          pltpu.SemaphoreType.DMA,
      ],
  )
  def prop_kernel(src_hbm, dst_hbm, u_hbm, out_hbm,
                  src_v, dst_v, rows_v, acc, sem0, sem1, sem2):
    c = lax.axis_index("c")
    s = lax.axis_index("s")
    sems = (sem0, sem1, sem2)
    pltpu.sync_copy(src_hbm.at[pl.ds(c * _E + s * _EPP, _EPP)], src_v)
    pltpu.sync_copy(dst_hbm.at[pl.ds(s * _EPP, _EPP)], dst_v)

    zero16 = jnp.zeros((16,), jnp.float32)

    def fill(i, carry):
      for k in range(dwh // 16):
        rows_v[2, i, pl.ds(k * 16, 16)] = zero16
      return carry

    lax.fori_loop(0, _ZB, fill, 0)

    for k in range(_RPT // _ZB):
      pltpu.sync_copy(rows_v.at[2, pl.ds(0, _ZB)],
                      acc.at[pl.ds(s * _RPT + k * _ZB, _ZB)])

    def _src(j):
      return src_v.at[pl.ds(j * _CHP, _CHP)]

    def _dst(j):
      return dst_v.at[pl.ds(j * _CHP, _CHP)]

    def _gather(j, b):
      pltpu.async_copy(u_hbm.at[_src(j)], rows_v.at[b], sems[b])

    def _wait(j, b):
      pltpu.make_async_copy(u_hbm.at[_src(j)], rows_v.at[b], sems[b]).wait()

    def _scat(j, b):
      pltpu.sync_copy(rows_v.at[b], acc.at[_dst(j)], add=True)

    _gather(0, 0)
    _gather(1, 1)
    plsc.subcore_barrier()

    def body(k, carry):
      j = 6 * k
      for t in range(6):
        jt = j + t
        _gather(jt + 2, (t + 2) % 3)
        _wait(jt, t % 3)
        _scat(jt, t % 3)
      return carry

    lax.fori_loop(0, (_NCHP - 4) // 6, body, 0)
    _j = _NCHP - 4
    _gather(_j + 2, (_j + 2) % 3)
    _wait(_j, _j % 3)
    _scat(_j, _j % 3)
    _gather(_j + 3, (_j + 3) % 3)
    _wait(_j + 1, (_j + 1) % 3)
    _scat(_j + 1, (_j + 1) % 3)
    _wait(_j + 2, (_j + 2) % 3)
    _scat(_j + 2, (_j + 2) % 3)
    _wait(_j + 3, (_j + 3) % 3)
    _scat(_j + 3, (_j + 3) % 3)
    plsc.subcore_barrier()
    pltpu.sync_copy(acc.at[pl.ds(s * _RPT, _RPT)],
                    out_hbm.at[pl.ds(s * _RPT, _RPT), pl.ds(c * dwh, dwh)])

  return prop_kernel


_deg_call = _make_deg_kernel()
_prop128 = _make_prop_kernel(_D)
_prop64 = _make_prop_kernel(_DLAST)


_R = 2000
_GRID = _N // _R


def _tc1_body(parts_ref, x_ref, w_ref, u0_ref, norm_ref):
  deg = parts_ref[0, :, 0:1] + parts_ref[1, :, 0:1]
  norm = lax.rsqrt(jnp.maximum(deg, 1.0))
  t = jnp.dot(x_ref[...], w_ref[...],
              preferred_element_type=jnp.float32) * norm
  u0_ref[...] = t
  norm_ref[...] = norm


def _tc1(parts, x, w0):
  return pl.pallas_call(
      _tc1_body,
      grid=(_GRID,),
      in_specs=[
          pl.BlockSpec((_NC, _R, 16), lambda i: (0, i, 0)),
          pl.BlockSpec((_R, _D), lambda i: (i, 0)),
          pl.BlockSpec((_D, _D), lambda i: (0, 0)),
      ],
      out_specs=[
          pl.BlockSpec((_R, _D), lambda i: (i, 0)),
          pl.BlockSpec((_R, 1), lambda i: (i, 0)),
      ],
      out_shape=[
          jax.ShapeDtypeStruct((_N, _D), jnp.float32),
          jax.ShapeDtypeStruct((_N, 1), jnp.float32),
      ],
  )(parts, x, w0)


def _mid_body(agg_ref, norm_ref, b_ref, w_ref, out_ref):
  n = norm_ref[...]
  h = jnp.maximum(agg_ref[...] * n + b_ref[...], 0.0)
  out_ref[...] = jnp.dot(h, w_ref[...], preferred_element_type=jnp.float32) * n


def _tc_mid(agg, norm, b, w, dw_out):
  return pl.pallas_call(
      _mid_body,
      grid=(_GRID,),
      in_specs=[
          pl.BlockSpec((_R, _D), lambda i: (i, 0)),
          pl.BlockSpec((_R, 1), lambda i: (i, 0)),
          pl.BlockSpec((1, _D), lambda i: (0, 0)),
          pl.BlockSpec((_D, dw_out), lambda i: (0, 0)),
      ],
      out_specs=pl.BlockSpec((_R, dw_out), lambda i: (i, 0)),
      out_shape=jax.ShapeDtypeStruct((_N, dw_out), jnp.float32),
  )(agg, norm, b, w)


def _fin_body(agg_ref, norm_ref, b_ref, out_ref):
  out_ref[...] = agg_ref[...] * norm_ref[...] + b_ref[...]


def _tc_fin(agg, norm, b):
  return pl.pallas_call(
      _fin_body,
      grid=(_GRID,),
      in_specs=[
          pl.BlockSpec((_R, _DLAST), lambda i: (i, 0)),
          pl.BlockSpec((_R, 1), lambda i: (i, 0)),
          pl.BlockSpec((1, _DLAST), lambda i: (0, 0)),
      ],
      out_specs=pl.BlockSpec((_R, _DLAST), lambda i: (i, 0)),
      out_shape=jax.ShapeDtypeStruct((_N, _DLAST), jnp.float32),
  )(agg, norm, b)


def kernel(x, edge_index, W0, b0, W1, b1, W2, b2):
  src = edge_index[0]
  dst = edge_index[1]

  src2x = jnp.concatenate([src * 2, src * 2 + 1])
  dstf = dst.reshape(_E)

  w2p = jnp.zeros((_D, _DLAST), jnp.float32).at[:, :_NCLS].set(W2)
  b2p = jnp.zeros((_DLAST,), jnp.float32).at[:_NCLS].set(b2)

  deg_parts = _deg_call(dstf)
  u0, norm = _tc1(deg_parts, x, W0)
  agg0 = _prop128(src2x, dstf, u0.reshape(2 * _N, _D // 2))
  u1 = _tc_mid(agg0, norm,
               b0.reshape(1, _D), W1, _D)
  agg1 = _prop128(src2x, dstf, u1.reshape(2 * _N, _D // 2))
  u2 = _tc_mid(agg1, norm,
               b1.reshape(1, _D), w2p, _DLAST)
  agg2 = _prop64(src2x, dstf, u2.reshape(2 * _N, _DLAST // 2))
  out = _tc_fin(agg2, norm,
                b2p.reshape(1, _DLAST))
  return out[:, :_NCLS]


# speedup vs baseline: 15.8055x; 1.0034x over previous
import functools

import jax
import jax.numpy as jnp
from jax import lax
from jax.experimental import pallas as pl
from jax.experimental.pallas import tpu as pltpu
from jax.experimental.pallas import tpu_sc as plsc

_N = 10000
_E = 320000
_D = 128
_DLAST = 64
_NCLS = 40

_NC = 2
_NS = 16
_NT = _NC * _NS
_CHP = 200
_CHD = 400
_EPD = _E // _NT
_EPP = _E // _NS
_NCHD = _EPD // _CHD
_NCHP = _EPP // _CHP
_RPT = 640
_NPAD = _NS * _RPT
_ZB = 128

_mesh = lambda: plsc.VectorSubcoreMesh(core_axis_name="c", subcore_axis_name="s")


def _make_deg_kernel():
  @functools.partial(
      pl.kernel,
      mesh=_mesh(),
      compiler_params=pltpu.CompilerParams(use_tc_tiling_on_sc=False),
      out_type=jax.ShapeDtypeStruct((_NC, _NPAD, 16), jnp.float32),
      scratch_types=[
          pltpu.VMEM((_EPD,), jnp.int32),
          pltpu.VMEM((_CHD, 16), jnp.float32),
          pltpu.VMEM((_ZB, 16), jnp.float32),
          pltpu.VMEM_SHARED((_NPAD, 16), jnp.float32),
          pltpu.SemaphoreType.DMA,
      ],
  )
  def deg_kernel(dst_hbm, out_hbm, dst_v, ones_v, zbuf_v, acc, sem):
    c = lax.axis_index("c")
    s = lax.axis_index("s")
    wid = c * _NS + s
    pltpu.sync_copy(dst_hbm.at[pl.ds(wid * _EPD, _EPD)], dst_v)

    one16 = jnp.ones((16,), jnp.float32)
    zero16 = jnp.zeros((16,), jnp.float32)

    def fill_ones(i, carry):
      ones_v[i, :] = one16
      return carry

    lax.fori_loop(0, _CHD, fill_ones, 0)

    def fill_zeros(i, carry):
      zbuf_v[i, :] = zero16
      return carry

    lax.fori_loop(0, _ZB, fill_zeros, 0)

    for k in range(_RPT // _ZB):
      pltpu.sync_copy(zbuf_v, acc.at[pl.ds(s * _RPT + k * _ZB, _ZB)])
    plsc.subcore_barrier()

    def body(j, carry):
      pltpu.async_copy(ones_v, acc.at[dst_v.at[pl.ds(j * _CHD, _CHD)]],
                       sem, add=True)
      return carry

    lax.fori_loop(0, _NCHD, body, 0)

    def drain(j, carry):
      pltpu.make_async_copy(ones_v, acc.at[dst_v.at[pl.ds(j * _CHD, _CHD)]],
                            sem).wait()
      return carry

    lax.fori_loop(0, _NCHD, drain, 0)
    plsc.subcore_barrier()
    pltpu.sync_copy(acc.at[pl.ds(s * _RPT, _RPT)],
                    out_hbm.at[c, pl.ds(s * _RPT, _RPT)])

  return deg_kernel


def _make_prop_kernel(dw):
  dwh = dw // 2

  @functools.partial(
      pl.kernel,
      mesh=_mesh(),
      compiler_params=pltpu.CompilerParams(use_tc_tiling_on_sc=False),
      out_type=jax.ShapeDtypeStruct((_NPAD, dw), jnp.float32),
      scratch_types=[
          pltpu.VMEM((_EPP,), jnp.int32),
          pltpu.VMEM((_EPP,), jnp.int32),
          pltpu.VMEM((3, _CHP, dwh), jnp.float32),
          pltpu.VMEM_SHARED((_NPAD, dwh), jnp.float32),
          pltpu.SemaphoreType.DMA,
          pltpu.SemaphoreType.DMA,
          pltpu.SemaphoreType.DMA,
      ],
  )
  def prop_kernel(src_hbm, dst_hbm, u_hbm, out_hbm,
                  src_v, dst_v, rows_v, acc, sem0, sem1, sem2):
    c = lax.axis_index("c")
    s = lax.axis_index("s")
    sems = (sem0, sem1, sem2)
    pltpu.sync_copy(src_hbm.at[pl.ds(c * _E + s * _EPP, _EPP)], src_v)
    pltpu.sync_copy(dst_hbm.at[pl.ds(s * _EPP, _EPP)], dst_v)

    zero16 = jnp.zeros((16,), jnp.float32)

    def fill(i, carry):
      for k in range(dwh // 16):
        rows_v[2, i, pl.ds(k * 16, 16)] = zero16
      return carry

    lax.fori_loop(0, _ZB, fill, 0)

    for k in range(_RPT // _ZB):
      pltpu.sync_copy(rows_v.at[2, pl.ds(0, _ZB)],
                      acc.at[pl.ds(s * _RPT + k * _ZB, _ZB)])

    def _src(j):
      return src_v.at[pl.ds(j * _CHP, _CHP)]

    def _dst(j):
      return dst_v.at[pl.ds(j * _CHP, _CHP)]

    def _gather(j, b):
      pltpu.async_copy(u_hbm.at[_src(j)], rows_v.at[b], sems[b])

    def _wait(j, b):
      pltpu.make_async_copy(u_hbm.at[_src(j)], rows_v.at[b], sems[b]).wait()

    def _scat(j, b):
      pltpu.sync_copy(rows_v.at[b], acc.at[_dst(j)], add=True)

    _gather(0, 0)
    _gather(1, 1)
    plsc.subcore_barrier()

    def body(k, carry):
      j = 6 * k
      for t in range(6):
        jt = j + t
        _gather(jt + 2, (t + 2) % 3)
        _wait(jt, t % 3)
        _scat(jt, t % 3)
      return carry

    lax.fori_loop(0, (_NCHP - 4) // 6, body, 0)
    _j = _NCHP - 4
    _gather(_j + 2, (_j + 2) % 3)
    _wait(_j, _j % 3)
    _scat(_j, _j % 3)
    _gather(_j + 3, (_j + 3) % 3)
    _wait(_j + 1, (_j + 1) % 3)
    _scat(_j + 1, (_j + 1) % 3)
    _wait(_j + 2, (_j + 2) % 3)
    _scat(_j + 2, (_j + 2) % 3)
    _wait(_j + 3, (_j + 3) % 3)
    _scat(_j + 3, (_j + 3) % 3)
    plsc.subcore_barrier()
    pltpu.sync_copy(acc.at[pl.ds(s * _RPT, _RPT)],
                    out_hbm.at[pl.ds(s * _RPT, _RPT), pl.ds(c * dwh, dwh)])

  return prop_kernel


_deg_call = _make_deg_kernel()
_prop128 = _make_prop_kernel(_D)
_prop64 = _make_prop_kernel(_DLAST)


_R = 2000
_GRID = _N // _R


def _tc1_body(parts_ref, x_ref, w_ref, u0_ref, norm_ref, norm2_ref):
  deg = parts_ref[0, :, 0:1] + parts_ref[1, :, 0:1]
  norm = lax.rsqrt(jnp.maximum(deg, 1.0))
  t = jnp.dot(x_ref[...], w_ref[...],
              preferred_element_type=jnp.float32) * norm
  u0_ref[...] = t
  norm_ref[...] = norm
  norm2_ref[...] = norm.reshape(_R // 2, 2)


def _tc1(parts, x, w0):
  return pl.pallas_call(
      _tc1_body,
      grid=(_GRID,),
      in_specs=[
          pl.BlockSpec((_NC, _R, 16), lambda i: (0, i, 0)),
          pl.BlockSpec((_R, _D), lambda i: (i, 0)),
          pl.BlockSpec((_D, _D), lambda i: (0, 0)),
      ],
      out_specs=[
          pl.BlockSpec((_R, _D), lambda i: (i, 0)),
          pl.BlockSpec((_R, 1), lambda i: (i, 0)),
          pl.BlockSpec((_R // 2, 2), lambda i: (i, 0)),
      ],
      out_shape=[
          jax.ShapeDtypeStruct((_N, _D), jnp.float32),
          jax.ShapeDtypeStruct((_N, 1), jnp.float32),
          jax.ShapeDtypeStruct((_N // 2, 2), jnp.float32),
      ],
  )(parts, x, w0)


def _mid_body(agg_ref, norm_ref, b_ref, w_ref, out_ref):
  n = norm_ref[...]
  h = jnp.maximum(agg_ref[...] * n + b_ref[...], 0.0)
  out_ref[...] = jnp.dot(h, w_ref[...], preferred_element_type=jnp.float32) * n


def _mid_pair_body(agg_ref, norm_ref, b_ref, w_ref, out_ref):
  n = norm_ref[...]
  h = jnp.maximum(agg_ref[...] * n + b_ref[...], 0.0)
  t = jnp.dot(h, w_ref[...], preferred_element_type=jnp.float32) * n
  out_ref[...] = t.reshape(_R // 2, 2 * t.shape[1])


def _tc_mid_pair(agg, norm, b, w, dw_out):
  return pl.pallas_call(
      _mid_pair_body,
      grid=(_GRID,),
      in_specs=[
          pl.BlockSpec((_R, _D), lambda i: (i, 0)),
          pl.BlockSpec((_R, 1), lambda i: (i, 0)),
          pl.BlockSpec((1, _D), lambda i: (0, 0)),
          pl.BlockSpec((_D, dw_out), lambda i: (0, 0)),
      ],
      out_specs=pl.BlockSpec((_R // 2, 2 * dw_out), lambda i: (i, 0)),
      out_shape=jax.ShapeDtypeStruct((_N // 2, 2 * dw_out), jnp.float32),
  )(agg, norm, b, w)


def _tc_mid(agg, norm, b, w, dw_out):
  return pl.pallas_call(
      _mid_body,
      grid=(_GRID,),
      in_specs=[
          pl.BlockSpec((_R, _D), lambda i: (i, 0)),
          pl.BlockSpec((_R, 1), lambda i: (i, 0)),
          pl.BlockSpec((1, _D), lambda i: (0, 0)),
          pl.BlockSpec((_D, dw_out), lambda i: (0, 0)),
      ],
      out_specs=pl.BlockSpec((_R, dw_out), lambda i: (i, 0)),
      out_shape=jax.ShapeDtypeStruct((_N, dw_out), jnp.float32),
  )(agg, norm, b, w)


def _fin_body(agg_ref, norm2_ref, b_ref, out_ref):
  a = agg_ref[...]
  n2 = norm2_ref[...]
  n = jnp.broadcast_to(n2[:, :, None], (_R // 2, 2, _DLAST))
  out_ref[...] = a * n.reshape(_R // 2, 2 * _DLAST) + b_ref[...]


def _tc_fin(agg, norm2, b2):
  return pl.pallas_call(
      _fin_body,
      grid=(_GRID,),
      in_specs=[
          pl.BlockSpec((_R // 2, 2 * _DLAST), lambda i: (i, 0)),
          pl.BlockSpec((_R // 2, 2), lambda i: (i, 0)),
          pl.BlockSpec((1, 2 * _DLAST), lambda i: (0, 0)),
      ],
      out_specs=pl.BlockSpec((_R // 2, 2 * _DLAST), lambda i: (i, 0)),
      out_shape=jax.ShapeDtypeStruct((_N // 2, 2 * _DLAST), jnp.float32),
  )(agg, norm2, b2)


def kernel(x, edge_index, W0, b0, W1, b1, W2, b2):
  src = edge_index[0]
  dst = edge_index[1]

  src2x = jnp.concatenate([src * 2, src * 2 + 1])
  dstf = dst.reshape(_E)

  w2p = jnp.zeros((_D, _DLAST), jnp.float32).at[:, :_NCLS].set(W2)
  b2p = jnp.zeros((_DLAST,), jnp.float32).at[:_NCLS].set(b2)

  deg_parts = _deg_call(dstf)
  u0, norm, norm2 = _tc1(deg_parts, x, W0)
  agg0 = _prop128(src2x, dstf, u0.reshape(2 * _N, _D // 2))
  u1 = _tc_mid(agg0, norm, b0.reshape(1, _D), W1, _D)
  agg1 = _prop128(src2x, dstf, u1.reshape(2 * _N, _D // 2))
  u2 = _tc_mid(agg1, norm, b1.reshape(1, _D), w2p, _DLAST)
  agg2 = _prop64(src2x, dstf, u2.reshape(2 * _N, _DLAST // 2))
  b2pp = jnp.concatenate([b2p, b2p]).reshape(1, 2 * _DLAST)
  out = _tc_fin(agg2.reshape(_NPAD // 2, 2 * _DLAST), norm2,
                b2pp)
  return out.reshape(_N, _DLAST)[:, :_NCLS]


# speedup vs baseline: 16.7258x; 1.0582x over previous
import functools

import jax
import jax.numpy as jnp
from jax import lax
from jax.experimental import pallas as pl
from jax.experimental.pallas import tpu as pltpu
from jax.experimental.pallas import tpu_sc as plsc

_N = 10000
_E = 320000
_D = 128
_DLAST = 64
_NCLS = 40

_NC = 2
_NS = 16
_NT = _NC * _NS
_CHP = 200
_EPP = _E // _NS
_NBLKS = _E // 128
_NBLK = _NBLKS // _NT
_NBLKR = _NBLKS % _NT
_NCHP = _EPP // _CHP
_RPT = 640
_NPAD = _NS * _RPT
_ZB = 128

_mesh = lambda: plsc.VectorSubcoreMesh(core_axis_name="c", subcore_axis_name="s")


def _make_deg_kernel():
  @functools.partial(
      pl.kernel,
      mesh=_mesh(),
      compiler_params=pltpu.CompilerParams(use_tc_tiling_on_sc=False),
      out_type=jax.ShapeDtypeStruct((_NPAD, 128), jnp.float32),
      scratch_types=[
          pltpu.VMEM((_NBLK + 1, 2, 128), jnp.int32),
          pltpu.VMEM((128, 16), jnp.float32),
          pltpu.VMEM((_ZB, 16), jnp.float32),
          pltpu.VMEM_SHARED((_NPAD, 16), jnp.float32),
          pltpu.SemaphoreType.DMA,
      ],
  )
  def deg_kernel(ei3_hbm, out_hbm, idx_v, ones_v, zbuf_v, acc, sem):
    c = lax.axis_index("c")
    s = lax.axis_index("s")
    wid = c * _NS + s
    base = _NBLK * wid + jnp.minimum(wid, _NBLKR)
    nblk = _NBLK + jnp.where(wid < _NBLKR, 1, 0)
    cap = jnp.minimum(base, _NBLKS - (_NBLK + 1))
    shift = base - cap
    pltpu.sync_copy(ei3_hbm.at[pl.ds(cap, _NBLK + 1)], idx_v)

    one16 = jnp.ones((16,), jnp.float32)
    zero16 = jnp.zeros((16,), jnp.float32)

    def fill_ones(i, carry):
      ones_v[i, :] = one16
      return carry

    lax.fori_loop(0, 128, fill_ones, 0)

    def fill_zeros(i, carry):
      zbuf_v[i, :] = zero16
      return carry

    lax.fori_loop(0, _ZB, fill_zeros, 0)

    for k in range(_RPT // _ZB):
      pltpu.sync_copy(zbuf_v, acc.at[pl.ds(s * _RPT + k * _ZB, _ZB)])
    plsc.subcore_barrier()

    def body(j, carry):
      pltpu.async_copy(ones_v, acc.at[idx_v.at[j, 1]], sem, add=True)
      return carry

    lax.fori_loop(shift, shift + nblk, body, 0)

    def drain(j, carry):
      pltpu.make_async_copy(ones_v, acc.at[idx_v.at[j, 1]], sem).wait()
      return carry

    lax.fori_loop(shift, shift + nblk, drain, 0)
    plsc.subcore_barrier()
    pltpu.sync_copy(acc.at[pl.ds(s * _RPT, _RPT)],
                    out_hbm.at[pl.ds(s * _RPT, _RPT), pl.ds(c * 16, 16)])

  return deg_kernel


def _make_prop_kernel(dw):
  dwh = dw // 2

  @functools.partial(
      pl.kernel,
      mesh=_mesh(),
      compiler_params=pltpu.CompilerParams(use_tc_tiling_on_sc=False),
      out_type=jax.ShapeDtypeStruct((_NPAD, dw), jnp.float32),
      scratch_types=[
          pltpu.VMEM((_EPP,), jnp.int32),
          pltpu.VMEM((_EPP,), jnp.int32),
          pltpu.VMEM((3, _CHP, dwh), jnp.float32),
          pltpu.VMEM_SHARED((_NPAD, dwh), jnp.float32),
          pltpu.SemaphoreType.DMA,
          pltpu.SemaphoreType.DMA,
          pltpu.SemaphoreType.DMA,
      ],
  )
  def prop_kernel(src_hbm, dst_hbm, u_hbm, out_hbm,
                  src_v, dst_v, rows_v, acc, sem0, sem1, sem2):
    c = lax.axis_index("c")
    s = lax.axis_index("s")
    sems = (sem0, sem1, sem2)
    pltpu.sync_copy(src_hbm.at[pl.ds(c * _E + s * _EPP, _EPP)], src_v)
    pltpu.sync_copy(dst_hbm.at[pl.ds(s * _EPP, _EPP)], dst_v)

    zero16 = jnp.zeros((16,), jnp.float32)

    def fill(i, carry):
      for k in range(dwh // 16):
        rows_v[2, i, pl.ds(k * 16, 16)] = zero16
      return carry

    lax.fori_loop(0, _ZB, fill, 0)

    for k in range(_RPT // _ZB):
      pltpu.sync_copy(rows_v.at[2, pl.ds(0, _ZB)],
                      acc.at[pl.ds(s * _RPT + k * _ZB, _ZB)])

    def _src(j):
      return src_v.at[pl.ds(j * _CHP, _CHP)]

    def _dst(j):
      return dst_v.at[pl.ds(j * _CHP, _CHP)]

    def _gather(j, b):
      pltpu.async_copy(u_hbm.at[_src(j)], rows_v.at[b], sems[b])

    def _wait(j, b):
      pltpu.make_async_copy(u_hbm.at[_src(j)], rows_v.at[b], sems[b]).wait()

    def _scat(j, b):
      pltpu.sync_copy(rows_v.at[b], acc.at[_dst(j)], add=True)

    _gather(0, 0)
    _gather(1, 1)
    plsc.subcore_barrier()

    def body(k, carry):
      j = 6 * k
      for t in range(6):
        jt = j + t
        _gather(jt + 2, (t + 2) % 3)
        _wait(jt, t % 3)
        _scat(jt, t % 3)
      return carry

    lax.fori_loop(0, (_NCHP - 4) // 6, body, 0)
    _j = _NCHP - 4
    _gather(_j + 2, (_j + 2) % 3)
    _wait(_j, _j % 3)
    _scat(_j, _j % 3)
    _gather(_j + 3, (_j + 3) % 3)
    _wait(_j + 1, (_j + 1) % 3)
    _scat(_j + 1, (_j + 1) % 3)
    _wait(_j + 2, (_j + 2) % 3)
    _scat(_j + 2, (_j + 2) % 3)
    _wait(_j + 3, (_j + 3) % 3)
    _scat(_j + 3, (_j + 3) % 3)
    plsc.subcore_barrier()
    pltpu.sync_copy(acc.at[pl.ds(s * _RPT, _RPT)],
                    out_hbm.at[pl.ds(s * _RPT, _RPT), pl.ds(c * dwh, dwh)])

  return prop_kernel


_deg_call = _make_deg_kernel()
_prop128 = _make_prop_kernel(_D)
_prop64 = _make_prop_kernel(_DLAST)


_R = 2000
_GRID = _N // _R


def _tc1_body(parts_ref, x_ref, w_ref, u0_ref, norm_ref, norm2_ref):
  deg = parts_ref[:, 0:1] + parts_ref[:, 16:17]
  norm = lax.rsqrt(jnp.maximum(deg, 1.0))
  t = jnp.dot(x_ref[...], w_ref[...],
              preferred_element_type=jnp.float32) * norm
  u0_ref[...] = t
  norm_ref[...] = norm
  norm2_ref[...] = norm.reshape(_R // 2, 2)


def _tc1(parts, x, w0):
  return pl.pallas_call(
      _tc1_body,
      grid=(_GRID,),
      in_specs=[
          pl.BlockSpec((_R, 128), lambda i: (i, 0)),
          pl.BlockSpec((_R, _D), lambda i: (i, 0)),
          pl.BlockSpec((_D, _D), lambda i: (0, 0)),
      ],
      out_specs=[
          pl.BlockSpec((_R, _D), lambda i: (i, 0)),
          pl.BlockSpec((_R, 1), lambda i: (i, 0)),
          pl.BlockSpec((_R // 2, 2), lambda i: (i, 0)),
      ],
      out_shape=[
          jax.ShapeDtypeStruct((_N, _D), jnp.float32),
          jax.ShapeDtypeStruct((_N, 1), jnp.float32),
          jax.ShapeDtypeStruct((_N // 2, 2), jnp.float32),
      ],
  )(parts, x, w0)


def _mid_body(agg_ref, norm_ref, b_ref, w_ref, out_ref):
  n = norm_ref[...]
  h = jnp.maximum(agg_ref[...] * n + b_ref[...], 0.0)
  out_ref[...] = jnp.dot(h, w_ref[...], preferred_element_type=jnp.float32) * n


def _mid_pair_body(agg_ref, norm_ref, b_ref, w_ref, out_ref):
  n = norm_ref[...]
  h = jnp.maximum(agg_ref[...] * n + b_ref[...], 0.0)
  t = jnp.dot(h, w_ref[...], preferred_element_type=jnp.float32) * n
  out_ref[...] = t.reshape(_R // 2, 2 * t.shape[1])


def _tc_mid_pair(agg, norm, b, w, dw_out):
  return pl.pallas_call(
      _mid_pair_body,
      grid=(_GRID,),
      in_specs=[
          pl.BlockSpec((_R, _D), lambda i: (i, 0)),
          pl.BlockSpec((_R, 1), lambda i: (i, 0)),
          pl.BlockSpec((1, _D), lambda i: (0, 0)),
          pl.BlockSpec((_D, dw_out), lambda i: (0, 0)),
      ],
      out_specs=pl.BlockSpec((_R // 2, 2 * dw_out), lambda i: (i, 0)),
      out_shape=jax.ShapeDtypeStruct((_N // 2, 2 * dw_out), jnp.float32),
  )(agg, norm, b, w)


def _tc_mid(agg, norm, b, w, dw_out):
  return pl.pallas_call(
      _mid_body,
      grid=(_GRID,),
      in_specs=[
          pl.BlockSpec((_R, _D), lambda i: (i, 0)),
          pl.BlockSpec((_R, 1), lambda i: (i, 0)),
          pl.BlockSpec((1, _D), lambda i: (0, 0)),
          pl.BlockSpec((_D, dw_out), lambda i: (0, 0)),
      ],
      out_specs=pl.BlockSpec((_R, dw_out), lambda i: (i, 0)),
      out_shape=jax.ShapeDtypeStruct((_N, dw_out), jnp.float32),
  )(agg, norm, b, w)


def _fin_body(agg_ref, norm2_ref, b_ref, out_ref):
  a = agg_ref[...]
  n2 = norm2_ref[...]
  n = jnp.broadcast_to(n2[:, :, None], (_R // 2, 2, _DLAST))
  out_ref[...] = a * n.reshape(_R // 2, 2 * _DLAST) + b_ref[...]


def _tc_fin(agg, norm2, b2):
  return pl.pallas_call(
      _fin_body,
      grid=(_GRID,),
      in_specs=[
          pl.BlockSpec((_R // 2, 2 * _DLAST), lambda i: (i, 0)),
          pl.BlockSpec((_R // 2, 2), lambda i: (i, 0)),
          pl.BlockSpec((1, 2 * _DLAST), lambda i: (0, 0)),
      ],
      out_specs=pl.BlockSpec((_R // 2, 2 * _DLAST), lambda i: (i, 0)),
      out_shape=jax.ShapeDtypeStruct((_N // 2, 2 * _DLAST), jnp.float32),
  )(agg, norm2, b2)


def kernel(x, edge_index, W0, b0, W1, b1, W2, b2):
  src = edge_index[0]
  dst = edge_index[1]

  src2x = jnp.concatenate([src * 2, src * 2 + 1])
  dstf = dst.reshape(_E)

  w2p = jnp.zeros((_D, _DLAST), jnp.float32).at[:, :_NCLS].set(W2)
  b2p = jnp.zeros((_DLAST,), jnp.float32).at[:_NCLS].set(b2)

  ei3 = edge_index.reshape(2, _NBLKS, 128).transpose(1, 0, 2)
  deg_parts = _deg_call(ei3)
  u0, norm, norm2 = _tc1(deg_parts, x, W0)
  agg0 = _prop128(src2x, dstf, u0.reshape(2 * _N, _D // 2))
  u1 = _tc_mid(agg0, norm, b0.reshape(1, _D), W1, _D)
  agg1 = _prop128(src2x, dstf, u1.reshape(2 * _N, _D // 2))
  u2 = _tc_mid(agg1, norm, b1.reshape(1, _D), w2p, _DLAST)
  agg2 = _prop64(src2x, dstf, u2.reshape(2 * _N, _DLAST // 2))
  b2pp = jnp.concatenate([b2p, b2p]).reshape(1, 2 * _DLAST)
  out = _tc_fin(agg2.reshape(_NPAD // 2, 2 * _DLAST), norm2,
                b2pp)
  return out.reshape(_N, _DLAST)[:, :_NCLS]
